# Initial kernel scaffold; baseline (speedup 1.0000x reference)
import functools

import jax
import jax.numpy as jnp
from jax import lax
from jax.experimental import pallas as pl
from jax.experimental.pallas import tpu as pltpu
from jax.experimental.pallas import tpu_sc as plsc

_NW = 32
_WIN = 128
_L = 16


def _make_sc_sampler(NP, C, H, W, P, chunk, nwin):
    mesh = plsc.VectorSubcoreMesh(core_axis_name="c", subcore_axis_name="s")

    @functools.partial(
        pl.kernel,
        mesh=mesh,
        out_type=jax.ShapeDtypeStruct((NP, C), jnp.float32),
        scratch_types=[
            pltpu.VMEM((_WIN,), jnp.float32),
            pltpu.VMEM((_WIN,), jnp.float32),
            pltpu.VMEM((4, _WIN), jnp.int32),
            pltpu.VMEM((4, _WIN), jnp.float32),
            pltpu.VMEM((4, _WIN, C), jnp.float32),
            pltpu.VMEM((_WIN, C), jnp.float32),
            pltpu.SemaphoreType.DMA,
        ],
    )
    def sampler(xt_hbm, gx_hbm, gy_hbm, out_hbm, gxv, gyv, idxv, wv, gv, ov,
                sem):
        wid = lax.axis_index("s") * 2 + lax.axis_index("c")
        base = wid * chunk
        nbase = (base // P) * P

        @pl.loop(0, nwin)
        def _window(w):
            start = base + w * _WIN
            pltpu.sync_copy(gx_hbm.at[pl.ds(start, _WIN)], gxv)
            pltpu.sync_copy(gy_hbm.at[pl.ds(start, _WIN)], gyv)

            @pl.loop(0, _WIN, step=_L)
            def _prep(i):
                s = pl.ds(i, _L)
                ix = (gxv[s] + 1.0) * (W * 0.5) - 0.5
                iy = (gyv[s] + 1.0) * (H * 0.5) - 0.5
                ix0 = (ix + 1024.0).astype(jnp.int32) - 1024
                iy0 = (iy + 1024.0).astype(jnp.int32) - 1024
                wx1 = ix - ix0.astype(jnp.float32)
                wy1 = iy - iy0.astype(jnp.float32)
                wx0 = 1.0 - wx1
                wy0 = 1.0 - wy1
                vx0 = (ix0 >= 0) & (ix0 <= W - 1)
                vx1 = (ix0 >= -1) & (ix0 <= W - 2)
                vy0 = (iy0 >= 0) & (iy0 <= H - 1)
                vy1 = (iy0 >= -1) & (iy0 <= H - 2)
                cx0 = jnp.clip(ix0, 0, W - 1)
                cx1 = jnp.clip(ix0 + 1, 0, W - 1)
                r0 = jnp.clip(iy0, 0, H - 1) * W + nbase
                r1 = jnp.clip(iy0 + 1, 0, H - 1) * W + nbase
                idxv[0, s] = r0 + cx0
                idxv[1, s] = r0 + cx1
                idxv[2, s] = r1 + cx0
                idxv[3, s] = r1 + cx1
                zero = jnp.zeros((_L,), jnp.float32)
                wv[0, s] = jnp.where(vx0 & vy0, wx0 * wy0, zero)
                wv[1, s] = jnp.where(vx1 & vy0, wx1 * wy0, zero)
                wv[2, s] = jnp.where(vx0 & vy1, wx0 * wy1, zero)
                wv[3, s] = jnp.where(vx1 & vy1, wx1 * wy1, zero)

            cps = [pltpu.async_copy(xt_hbm.at[idxv.at[k]], gv.at[k], sem)
                   for k in range(4)]
            for cp in cps:
                cp.wait()

            @pl.loop(0, _WIN)
            def _interp(p):
                w00 = wv[0, p]
                w01 = wv[1, p]
                w10 = wv[2, p]
                w11 = wv[3, p]
                for ch in range(C // _L):
                    c = pl.ds(ch * _L, _L)
                    ov[p, c] = (gv[0, p, c] * w00 + gv[1, p, c] * w01
                                + gv[2, p, c] * w10 + gv[3, p, c] * w11)

            pltpu.sync_copy(ov, out_hbm.at[pl.ds(start, _WIN)])

    return sampler


def kernel(x, grid):
    N, C, H, W = x.shape
    P = H * W
    NP = N * P
    chunk = NP // _NW
    nwin = chunk // _WIN
    xt = jnp.transpose(x, (0, 2, 3, 1)).reshape(NP, C)
    gx = grid[..., 0].reshape(NP)
    gy = grid[..., 1].reshape(NP)
    out_t = _make_sc_sampler(NP, C, H, W, P, chunk, nwin)(xt, gx, gy)
    return jnp.transpose(out_t.reshape(N, H, W, C), (0, 3, 1, 2))


# speedup vs baseline: 1.1237x; 1.1237x over previous
import functools

import jax
import jax.numpy as jnp
from jax import lax
from jax.experimental import pallas as pl
from jax.experimental.pallas import tpu as pltpu
from jax.experimental.pallas import tpu_sc as plsc

_NW = 32
_WIN = 128
_L = 16


def _make_sc_sampler(NP, C, H, W, P, chunk, nwin):
    mesh = plsc.VectorSubcoreMesh(core_axis_name="c", subcore_axis_name="s")

    @functools.partial(
        pl.kernel,
        mesh=mesh,
        out_type=jax.ShapeDtypeStruct((NP, C), jnp.float32),
        compiler_params=pltpu.CompilerParams(use_tc_tiling_on_sc=False),
        scratch_types=[
            pltpu.VMEM((_WIN,), jnp.float32),
            pltpu.VMEM((_WIN,), jnp.float32),
            pltpu.VMEM((4, _WIN), jnp.int32),
            pltpu.VMEM((4, _WIN), jnp.float32),
            pltpu.VMEM((4, _WIN, C), jnp.float32),
            pltpu.VMEM((_WIN, C), jnp.float32),
            pltpu.SemaphoreType.DMA,
        ],
    )
    def sampler(xt_hbm, gx_hbm, gy_hbm, out_hbm, gxv, gyv, idxv, wv, gv, ov,
                sem):
        wid = lax.axis_index("s") * 2 + lax.axis_index("c")
        base = wid * chunk
        nbase = (base // P) * P

        @pl.loop(0, nwin)
        def _window(w):
            start = base + w * _WIN
            pltpu.sync_copy(gx_hbm.at[pl.ds(start, _WIN)], gxv)
            pltpu.sync_copy(gy_hbm.at[pl.ds(start, _WIN)], gyv)

            @pl.loop(0, _WIN, step=_L)
            def _prep(i):
                s = pl.ds(i, _L)
                ix = (gxv[s] + 1.0) * (W * 0.5) - 0.5
                iy = (gyv[s] + 1.0) * (H * 0.5) - 0.5
                ix0 = (ix + 1024.0).astype(jnp.int32) - 1024
                iy0 = (iy + 1024.0).astype(jnp.int32) - 1024
                wx1 = ix - ix0.astype(jnp.float32)
                wy1 = iy - iy0.astype(jnp.float32)
                wx0 = 1.0 - wx1
                wy0 = 1.0 - wy1
                vx0 = (ix0 >= 0) & (ix0 <= W - 1)
                vx1 = (ix0 >= -1) & (ix0 <= W - 2)
                vy0 = (iy0 >= 0) & (iy0 <= H - 1)
                vy1 = (iy0 >= -1) & (iy0 <= H - 2)
                cx0 = jnp.clip(ix0, 0, W - 1)
                cx1 = jnp.clip(ix0 + 1, 0, W - 1)
                r0 = jnp.clip(iy0, 0, H - 1) * W + nbase
                r1 = jnp.clip(iy0 + 1, 0, H - 1) * W + nbase
                idxv[0, s] = r0 + cx0
                idxv[1, s] = r0 + cx1
                idxv[2, s] = r1 + cx0
                idxv[3, s] = r1 + cx1
                zero = jnp.zeros((_L,), jnp.float32)
                wv[0, s] = jnp.where(vx0 & vy0, wx0 * wy0, zero)
                wv[1, s] = jnp.where(vx1 & vy0, wx1 * wy0, zero)
                wv[2, s] = jnp.where(vx0 & vy1, wx0 * wy1, zero)
                wv[3, s] = jnp.where(vx1 & vy1, wx1 * wy1, zero)

            cps = [pltpu.async_copy(xt_hbm.at[idxv.at[k]], gv.at[k], sem)
                   for k in range(4)]
            for cp in cps:
                cp.wait()

            @pl.loop(0, _WIN, step=_L)
            def _interp(i):
                s = pl.ds(i, _L)
                wvec = [wv[k, s] for k in range(4)]
                for j in range(_L):
                    p = i + j
                    w00 = wvec[0][j]
                    w01 = wvec[1][j]
                    w10 = wvec[2][j]
                    w11 = wvec[3][j]
                    for ch in range(C // _L):
                        c = pl.ds(ch * _L, _L)
                        ov[p, c] = (gv[0, p, c] * w00 + gv[1, p, c] * w01
                                    + gv[2, p, c] * w10 + gv[3, p, c] * w11)

            pltpu.sync_copy(ov, out_hbm.at[pl.ds(start, _WIN)])

    return sampler


def kernel(x, grid):
    N, C, H, W = x.shape
    P = H * W
    NP = N * P
    chunk = NP // _NW
    nwin = chunk // _WIN
    xt = jnp.transpose(x, (0, 2, 3, 1)).reshape(NP, C)
    gx = grid[..., 0].reshape(NP)
    gy = grid[..., 1].reshape(NP)
    out_t = _make_sc_sampler(NP, C, H, W, P, chunk, nwin)(xt, gx, gy)
    return jnp.transpose(out_t.reshape(N, H, W, C), (0, 3, 1, 2))
